# baseline (speedup 1.0000x reference)
import functools

import jax
import jax.numpy as jnp
from jax import lax
from jax.experimental import pallas as pl
from jax.experimental.pallas import tpu as pltpu
from jax.experimental.pallas import tpu_sc as plsc

N = 16384
NCAT = 26
NNUM = 13
NCOL = NCAT + NNUM
VOCAB = 1000
C = 128
NW = 32
RPW = N // NW
RC = 4
IPC = RC * NCAT
NPC = RC * NNUM
NCH = RPW // RC
NSLOT = 3
LANES = 16

_mesh = plsc.VectorSubcoreMesh(core_axis_name="c", subcore_axis_name="s")


@functools.partial(
    pl.kernel,
    mesh=_mesh,
    out_type=jax.ShapeDtypeStruct((N * NCOL, C), jnp.float32),
    compiler_params=pltpu.CompilerParams(use_tc_tiling_on_sc=False),
    scratch_types=[
        pltpu.VMEM((NCH, IPC), jnp.int32),
        pltpu.VMEM((NCH, IPC), jnp.int32),
        pltpu.VMEM((NCH, NPC), jnp.int32),
        pltpu.VMEM((NNUM, RPW + LANES), jnp.float32),
        pltpu.VMEM((NNUM, C), jnp.float32),
        pltpu.VMEM((NNUM, C), jnp.float32),
        pltpu.VMEM((NSLOT, IPC, C), jnp.float32),
        pltpu.VMEM((NSLOT, NPC, C), jnp.float32),
        pltpu.SemaphoreType.DMA((NSLOT,)),
        pltpu.SemaphoreType.DMA((NSLOT,)),
    ],
)
def _encode(table_hbm, idx_hbm, dstc_hbm, dstn_hbm, fnum_hbm, w_hbm, b_hbm,
            out_hbm, idx_v, dstc_v, dstn_v, fnum_v, w_v, b_v, gbuf, nbuf,
            gsem, wsem):
    wid = lax.axis_index("s") * 2 + lax.axis_index("c")
    pltpu.sync_copy(idx_hbm.at[wid], idx_v)
    pltpu.sync_copy(dstc_hbm.at[wid], dstc_v)
    pltpu.sync_copy(dstn_hbm.at[wid], dstn_v)
    pltpu.sync_copy(fnum_hbm.at[wid], fnum_v)
    pltpu.sync_copy(w_hbm, w_v)
    pltpu.sync_copy(b_hbm, b_v)

    pltpu.async_copy(table_hbm.at[idx_v.at[0]], gbuf.at[0], gsem.at[0])
    pltpu.async_copy(table_hbm.at[idx_v.at[1]], gbuf.at[1], gsem.at[1])

    def chunk(c, carry):
        s = c % NSLOT

        def jbody(j, carry2):
            v16 = fnum_v[j, pl.ds(c * RC, LANES)]
            for r in range(RC):
                vb = jnp.full((LANES,), v16[r], dtype=jnp.float32)
                for k in range(C // LANES):
                    sl = pl.ds(k * LANES, LANES)
                    nbuf[s, r * NNUM + j, sl] = vb * w_v[j, sl] + b_v[j, sl]
            return carry2

        lax.fori_loop(0, NNUM, jbody, 0)

        pltpu.make_async_copy(table_hbm.at[idx_v.at[c]], gbuf.at[s],
                              gsem.at[s]).wait()

        @pl.when(c >= 1)
        def _():
            sp = (c + 2) % NSLOT
            pltpu.make_async_copy(gbuf.at[sp], out_hbm.at[dstc_v.at[c - 1]],
                                  wsem.at[sp]).wait()
            pltpu.make_async_copy(nbuf.at[sp], out_hbm.at[dstn_v.at[c - 1]],
                                  wsem.at[sp]).wait()

        @pl.when(c < NCH - 2)
        def _():
            sn = (c + 2) % NSLOT
            pltpu.async_copy(table_hbm.at[idx_v.at[c + 2]], gbuf.at[sn],
                             gsem.at[sn])

        pltpu.async_copy(gbuf.at[s], out_hbm.at[dstc_v.at[c]], wsem.at[s])
        pltpu.async_copy(nbuf.at[s], out_hbm.at[dstn_v.at[c]], wsem.at[s])
        return carry

    lax.fori_loop(0, NCH, chunk, 0)
    sl = (NCH - 1) % NSLOT
    pltpu.make_async_copy(gbuf.at[sl], out_hbm.at[dstc_v.at[NCH - 1]],
                          wsem.at[sl]).wait()
    pltpu.make_async_copy(nbuf.at[sl], out_hbm.at[dstn_v.at[NCH - 1]],
                          wsem.at[sl]).wait()


def kernel(feat_cat, feat_num, emb_tables, lin_weight, lin_bias, num_mean, num_std):
    table = emb_tables.reshape(NCAT * VOCAB, C)
    offs = jnp.arange(NCAT, dtype=jnp.int32) * VOCAB
    idx = (feat_cat.astype(jnp.int32) + offs[None, :]).reshape(NW, NCH, IPC)
    n_grid = jnp.arange(N, dtype=jnp.int32).reshape(NW, NCH, RC)
    dstc = (n_grid[..., None] * NCOL
            + jnp.arange(NCAT, dtype=jnp.int32)).reshape(NW, NCH, IPC)
    dstn = (n_grid[..., None] * NCOL + NCAT
            + jnp.arange(NNUM, dtype=jnp.int32)).reshape(NW, NCH, NPC)
    fnum = feat_num.reshape(NW, RPW, NNUM).transpose(0, 2, 1)
    fnum = jnp.pad(fnum, ((0, 0), (0, 0), (0, LANES)))
    inv = 1.0 / num_std
    w_eff = lin_weight * inv[:, None]
    b_eff = lin_bias - (num_mean * inv)[:, None] * lin_weight
    out = _encode(table, idx, dstc, dstn, fnum, w_eff, b_eff)
    return out.reshape(N, NCOL, C)
